# baseline (speedup 1.0000x reference)
import functools

import jax
import jax.numpy as jnp
from jax import lax
from jax.experimental import pallas as pl
from jax.experimental.pallas import tpu as pltpu
from jax.experimental.pallas import tpu_sc as plsc

_BATCH = 16384
_NF = 26
_VOCAB = 100000
_EMB = 16
_DENSE = 13
_OUTW = _NF * _EMB + _DENSE

_VPAD = 100096
_NG = 52
_NGO = 54
_NBB = _BATCH // 128
_NW = 32
_BPW = _NBB // _NW
_L = 16
_GSTRIDE = _VPAD * 8

_mesh = plsc.VectorSubcoreMesh(core_axis_name="c", subcore_axis_name="s")


@functools.partial(
    pl.kernel,
    mesh=_mesh,
    out_type=jax.ShapeDtypeStruct((_NGO, 128, 1024), jnp.float32),
    scratch_types=[
        pltpu.VMEM((_BPW, 4, 1024), jnp.int32),
        pltpu.VMEM((4, _BPW * 1024), jnp.int32),
        pltpu.VMEM((4, _BPW * 1024), jnp.float32),
        pltpu.VMEM((1024,), jnp.float32),
        pltpu.SemaphoreType.DMA((4,)),
        pltpu.SemaphoreType.DMA((4,)),
    ],
    compiler_params=pltpu.CompilerParams(
        use_tc_tiling_on_sc=False, needs_layout_passes=False),
)
def _fe(tab1d, xs4, xd4, out4, xs_v, idx_v, emb_v, dns_v, gsem, wsem):
    wid = lax.axis_index("s") * 2 + lax.axis_index("c")
    b0 = wid * _BPW

    for bi in range(_BPW):
        for gf in range(4):
            pltpu.sync_copy(xs4.at[gf, b0 + bi], xs_v.at[bi, gf])

    def build_idx(g, par):
        i = g >> 1
        gf = i >> 3
        f = i & 7
        base = g * _GSTRIDE
        for bi in range(_BPW):
            def chunk(p, carry, bi=bi):
                v = xs_v[bi, gf, pl.ds(f * 128 + p * _L, _L)]
                t = base + ((v >> 7) << 10) + (v & 127)
                for e in range(8):
                    idx_v[par, pl.ds(bi * 1024 + e * 128 + p * _L, _L)] = (
                        t + e * 128)
                return carry
            lax.fori_loop(0, 8, chunk, 0)

    def gather_desc(par):
        return pltpu.make_async_copy(
            tab1d.at[idx_v.at[par]], emb_v.at[par], gsem.at[par])

    def write_descs(g, par):
        return [pltpu.make_async_copy(
                    emb_v.at[par, pl.ds(bi * 1024, 1024)],
                    out4.at[g, b0 + bi], wsem.at[par])
                for bi in range(_BPW)]

    def loop_body(g, carry):
        par = g & 3

        @pl.when(g >= 4)
        def _():
            for d in write_descs(g - 4, par):
                d.wait()

        build_idx(g, par)
        gather_desc(par).start()

        @pl.when(g >= 1)
        def _():
            par1 = (g - 1) & 3
            gather_desc(par1).wait()
            for d in write_descs(g - 1, par1):
                d.start()
        return carry

    lax.fori_loop(0, _NG, loop_body, 0)

    last_par = (_NG - 1) & 3
    gather_desc(last_par).wait()
    for d in write_descs(_NG - 1, last_par):
        d.start()
    for t in range(_NG - 4, _NG - 1):
        for d in write_descs(t, t & 3):
            d.wait()

    for bi in range(_BPW):
        for k in range(2):
            pltpu.sync_copy(xd4.at[k, b0 + bi], dns_v)
            pltpu.sync_copy(dns_v, out4.at[_NG + k, b0 + bi])

    for d in write_descs(_NG - 1, last_par):
        d.wait()


def kernel(X_sparse, X_dense, tables):
    tabP = jnp.pad(tables, ((0, 0), (0, _VPAD - _VOCAB), (0, 0)))
    tab1d = (tabP.reshape(_NF, _VPAD // 128, 128, 2, 8)
             .transpose(0, 3, 1, 4, 2).reshape(_NF * _EMB * _VPAD))

    xs4 = (jnp.pad(X_sparse, ((0, 0), (0, 6))).astype(jnp.int32)
           .reshape(128, 128, 4, 8).transpose(2, 0, 3, 1)
           .reshape(4, 128, 1024))
    xd4 = (jnp.pad(X_dense, ((0, 0), (0, 3)))
           .reshape(128, 128, 2, 8).transpose(2, 0, 3, 1)
           .reshape(2, 128, 1024))

    o4 = _fe(tab1d, xs4, xd4)
    return (o4.reshape(_NGO, 128, 8, 128).transpose(1, 3, 0, 2)
            .reshape(_BATCH, _NGO * 8)[:, :_OUTW])
